# baseline (speedup 1.0000x reference)
import functools

import numpy as np
import jax
import jax.numpy as jnp
from jax import lax
from jax.experimental import pallas as pl
from jax.experimental.pallas import tpu as pltpu
from jax.experimental.pallas import tpu_sc as plsc

HIDDEN = 256
NUM_CLASSES = 8192
POS_LEN = 512
B = 50000

L = 16
NW = 32
BPT = 1568
CH = 112
NCH = BPT // CH


def _positional_table():
    dim, n = HIDDEN, POS_LEN
    enc = np.array([pos / np.power(10000, 2 * i / dim)
                    for pos in range(n) for i in range(dim)])
    enc[::2] = np.sin(enc[::2])
    enc[1::2] = np.cos(enc[1::2])
    pe = enc.reshape([n, dim]).astype(np.float32)
    return np.concatenate([np.zeros((1, dim), np.float32), pe], axis=0)


_POS = _positional_table()


def _body(idx0_hbm, idx1_hbm, w_hbm, pos_hbm, out_hbm,
          idxw_v, idxp_v, rows_w, rows_p, semw, semp):
    wid = lax.axis_index("s") * 2 + lax.axis_index("c")
    base = jnp.minimum(wid * BPT, B - BPT)

    def chunk(c, _):
        off = base + c * CH
        pltpu.sync_copy(idx0_hbm.at[pl.ds(off, CH)], idxw_v)
        pltpu.sync_copy(idx1_hbm.at[pl.ds(off, CH)], idxp_v)

        def clip(i, _):
            sl = pl.ds(i * L, L)
            idxp_v[sl] = jnp.minimum(idxp_v[sl], POS_LEN)
            return 0
        lax.fori_loop(0, CH // L, clip, 0, unroll=True)

        cw = pltpu.async_copy(w_hbm.at[idxw_v], rows_w, semw)
        cp = pltpu.async_copy(pos_hbm.at[idxp_v], rows_p, semp)
        cw.wait()
        cp.wait()

        def add_row(r, _):
            for j in range(HIDDEN // L):
                sl = pl.ds(j * L, L)
                rows_w[r, sl] = rows_w[r, sl] + rows_p[r, sl]
            return 0
        lax.fori_loop(0, CH, add_row, 0)

        pltpu.sync_copy(rows_w, out_hbm.at[pl.ds(off, CH)])
        return 0

    lax.fori_loop(0, NCH, chunk, 0)


@jax.jit
def _run(idx0, idx1, w, pos):
    mesh = plsc.VectorSubcoreMesh(core_axis_name="c", subcore_axis_name="s")
    f = pl.kernel(
        _body,
        out_type=jax.ShapeDtypeStruct((B, HIDDEN), jnp.float32),
        mesh=mesh,
        scratch_types=[
            pltpu.VMEM((CH,), jnp.int32),
            pltpu.VMEM((CH,), jnp.int32),
            pltpu.VMEM((CH, HIDDEN), jnp.float32),
            pltpu.VMEM((CH, HIDDEN), jnp.float32),
            pltpu.SemaphoreType.DMA,
            pltpu.SemaphoreType.DMA,
        ],
    )
    return f(idx0, idx1, w, pos)


def kernel(nodes, W):
    return _run(nodes[0], nodes[1], W, _POS)
